# baseline (speedup 1.0000x reference)
import jax
import jax.numpy as jnp
from jax.experimental import pallas as pl
from jax.experimental.pallas import tpu as pltpu

_D_MODEL = 4096
_RANK = 64
_EXPERTS = 64
_K = 8
_BLOCK = 1024


def _router_block_kernel(x_ref, u_ref, v_ref, b_ref, w_ref, i_ref):
    x = x_ref[...]
    xu = jnp.dot(x, u_ref[...], preferred_element_type=jnp.float32)
    s = jnp.dot(xu, v_ref[...], preferred_element_type=jnp.float32)
    s = s + b_ref[...]
    m = jnp.max(s, axis=-1, keepdims=True)
    e = jnp.exp(s - m)
    p = e / jnp.sum(e, axis=-1, keepdims=True)

    lane = jax.lax.broadcasted_iota(jnp.int32, p.shape, 1)
    col = jax.lax.broadcasted_iota(jnp.int32, (p.shape[0], _K), 1)
    pbits = jax.lax.bitcast_convert_type(p, jnp.int32)
    keys = jax.lax.bitcast_convert_type((pbits & ~63) | (63 - lane),
                                        jnp.float32)
    work = keys
    idx_out = jnp.zeros((p.shape[0], _K), jnp.int32)
    for j in range(_K):
        mx = jnp.max(work, axis=-1, keepdims=True)
        work = jnp.where(work == mx, -1.0, work)
        idx = 63 - (jax.lax.bitcast_convert_type(mx, jnp.int32) & 63)
        idx_out = jnp.where(col == j, idx, idx_out)

    w_ref[...] = jnp.where(work < 0.0, p, 0.0)
    i_ref[...] = idx_out


def kernel(prime_memory_output, U_route, V_route, routing_bias, top_k):
    tokens = prime_memory_output.shape[0]
    grid = (tokens // _BLOCK,)
    bias2d = routing_bias.reshape(1, _EXPERTS)
    weights, indices = pl.pallas_call(
        _router_block_kernel,
        grid=grid,
        in_specs=[
            pl.BlockSpec((_BLOCK, _D_MODEL), lambda i: (i, 0)),
            pl.BlockSpec((_D_MODEL, _RANK), lambda i: (0, 0)),
            pl.BlockSpec((_RANK, _EXPERTS), lambda i: (0, 0)),
            pl.BlockSpec((1, _EXPERTS), lambda i: (0, 0)),
        ],
        out_specs=[
            pl.BlockSpec((_BLOCK, _EXPERTS), lambda i: (i, 0)),
            pl.BlockSpec((_BLOCK, _K), lambda i: (i, 0)),
        ],
        out_shape=[
            jax.ShapeDtypeStruct((tokens, _EXPERTS), jnp.float32),
            jax.ShapeDtypeStruct((tokens, _K), jnp.int32),
        ],
        compiler_params=pltpu.CompilerParams(
            dimension_semantics=("parallel",),
        ),
    )(prime_memory_output, U_route, V_route, bias2d)
    return weights, indices
